# Initial kernel scaffold; baseline (speedup 1.0000x reference)
import jax
import jax.numpy as jnp
from jax.experimental import pallas as pl


def kernel(node_features, params, edge_index, map_entry_idx):
    raise NotImplementedError("write your pallas kernel here")


# speedup vs baseline: 1.0937x; 1.0937x over previous
import functools

import jax
import jax.numpy as jnp
from jax.experimental import pallas as pl


def _mm(x, W, b):
    N, K = x.shape
    M = W.shape[1]
    BR = 1000

    def body(x_ref, w_ref, b_ref, o_ref):
        o_ref[...] = jnp.dot(x_ref[...], w_ref[...],
                             preferred_element_type=jnp.float32) + b_ref[...]

    return pl.pallas_call(
        body,
        grid=(N // BR,),
        in_specs=[
            pl.BlockSpec((BR, K), lambda i: (i, 0)),
            pl.BlockSpec((K, M), lambda i: (0, 0)),
            pl.BlockSpec((M,), lambda i: (0,)),
        ],
        out_specs=pl.BlockSpec((BR, M), lambda i: (i, 0)),
        out_shape=jax.ShapeDtypeStruct((N, M), jnp.float32),
    )(x, W, b)


def kernel(node_features, params, edge_index, map_entry_idx):
    src = edge_index[0]
    dst = edge_index[1]
    n = node_features.shape[0]

    def seg_max(y):
        agg = jax.ops.segment_max(y[src], dst, num_segments=n)
        return jnp.where(jnp.isfinite(agg), agg, 0.0)

    h = node_features
    saved = h
    for i in range(3):
        y = _mm(h, params[f"Wm{i}"], params[f"bm{i}"])
        h = _mm(seg_max(y), params[f"Wd{i}"], params[f"bd{i}"])
    h = jnp.concatenate([saved, h], axis=-1)
    y = _mm(h, params["Wm3"], params["bm3"])
    h = _mm(seg_max(y), params["Wd3"], params["bd3"])
    saved = h
    for i in range(4, 7):
        y = _mm(h, params[f"Wm{i}"], params[f"bm{i}"])
        h = _mm(seg_max(y), params[f"Wd{i}"], params[f"bd{i}"])
    h = jnp.concatenate([saved, h], axis=-1)
    y = _mm(h, params["Wm7"], params["bm7"])
    agg = seg_max(y)
    row = jax.lax.dynamic_slice_in_dim(agg, map_entry_idx, 1, axis=0)
    x = row @ params["Wd7"] + params["bd7"]
    x = jax.nn.relu(x @ params["W1"] + params["b1"])
    x = (x @ params["W2"] + params["b2"])[0]
    return x
